# Initial kernel scaffold; baseline (speedup 1.0000x reference)
import jax
import jax.numpy as jnp
from jax.experimental import pallas as pl


def kernel(output, anchors, targets):
    raise NotImplementedError("write your pallas kernel here")


# speedup vs baseline: 3.8918x; 3.8918x over previous
import functools

import jax
import jax.numpy as jnp
from jax import lax
from jax.experimental import pallas as pl
from jax.experimental.pallas import tpu as pltpu
from jax.experimental.pallas import tpu_sc as plsc

_B, _A, _G, _C, _T = 32, 3, 52, 80, 100
_D = 5 + _C
_NCELL = _A * _G * _G
_TBL = 8192
_TPAD = 112
_NG = _TPAD // 16
_NC, _NS = 2, 16
_NROW = _B * _A * _G * _G

_mesh = plsc.VectorSubcoreMesh(
    core_axis_name="c", subcore_axis_name="s",
    num_cores=_NC, num_subcores=_NS)


@functools.partial(
    pl.kernel,
    out_type=(jax.ShapeDtypeStruct((_B, _TPAD, 128), jnp.float32),
              jax.ShapeDtypeStruct((_B, 8, _TPAD), jnp.float32)),
    mesh=_mesh,
    compiler_params=pltpu.CompilerParams(needs_layout_passes=False),
    scratch_types=(
        pltpu.VMEM((5, _TPAD), jnp.float32),
        pltpu.VMEM((6, 16), jnp.float32),
        pltpu.VMEM((_TBL,), jnp.int32),
        pltpu.VMEM((_NG, 16), jnp.int32),
        pltpu.VMEM((_TPAD,), jnp.int32),
        pltpu.VMEM((_TPAD, 128), jnp.float32),
        pltpu.VMEM((8, _TPAD), jnp.float32),
        pltpu.SemaphoreType.DMA,
    ),
)
def _sc_stage(outflat, tprep, ancrep, rows_out, meta_out,
              tloc, anc, table, cells, rowidx, rows, meta, sem):
    b = lax.axis_index("s") * _NC + lax.axis_index("c")
    pltpu.sync_copy(tprep.at[b], tloc)
    pltpu.sync_copy(ancrep, anc)

    def _zero(i, carry):
        for u in range(8):
            table[pl.ds(i * 128 + u * 16, 16)] = jnp.zeros((16,), jnp.int32)
        return carry
    lax.fori_loop(0, _TBL // 128, _zero, 0)

    lane = lax.iota(jnp.int32, 16)
    aw = [anc[2 * a, :] for a in range(_A)]
    ah = [anc[2 * a + 1, :] for a in range(_A)]

    for k in range(_NG):
        sl = pl.ds(k * 16, 16)
        t0 = tloc[0, sl]
        t1 = tloc[1, sl]
        t2 = tloc[2, sl]
        t3 = tloc[3, sl]
        t4 = tloc[4, sl]
        gxf = t0 * float(_G)
        gyf = t1 * float(_G)
        gx = gxf.astype(jnp.int32)
        gy = gyf.astype(jnp.int32)
        fx = gxf - gx.astype(jnp.float32)
        fy = gyf - gy.astype(jnp.float32)
        gw = jnp.abs(t2 - t0) * float(_G)
        gh = jnp.abs(t3 - t1) * float(_G)
        gprod = gw * gh
        best = jnp.zeros((16,), jnp.int32)
        bw = aw[0]
        bh = ah[0]
        inter = jnp.minimum(aw[0], gw) * jnp.minimum(ah[0], gh)
        biou = inter / (1e-08 + aw[0] * ah[0] + gprod - inter)
        for a in range(1, _A):
            inter = jnp.minimum(aw[a], gw) * jnp.minimum(ah[a], gh)
            iou = inter / (1e-08 + aw[a] * ah[a] + gprod - inter)
            upd = iou > biou
            best = jnp.where(upd, a, best)
            bw = jnp.where(upd, aw[a], bw)
            bh = jnp.where(upd, ah[a], bh)
            biou = jnp.maximum(biou, iou)

        locid = k * 16 + lane
        valid = locid < _T
        cell = (best * _G + gx) * _G + gy
        cellm = jnp.where(valid, cell, _NCELL + lane)
        rowg = ((b * _A + best) * _G + gx) * _G + gy
        cells[k, :] = cellm
        rowidx[sl] = jnp.where(valid, rowg, 0)
        meta[1, sl] = fx
        meta[2, sl] = fy
        meta[3, sl] = jnp.where(valid, gw / bw, 1.0)
        meta[4, sl] = jnp.where(valid, gh / bh, 1.0)
        meta[5, sl] = t4.astype(jnp.int32).astype(jnp.float32)

        ival = jnp.where(valid, locid + 1, 0)
        skey, sperm = plsc.sort_key_val(cellm * 16 + lane, lane)
        scell = lax.shift_right_logical(skey, 4)
        _, nxt = plsc.sort_key_val((lane + 15) & 15, scell)
        rep_sorted = jnp.where((scell != nxt) | (lane == 15), 1, 0)
        _, rep = plsc.sort_key_val(sperm, rep_sorted)
        old = plsc.load_gather(table, [cellm])
        plsc.store_scatter(table, [cellm], jnp.maximum(old, ival),
                           mask=valid & (rep > 0))

    gcp = pltpu.async_copy(outflat.at[rowidx], rows, sem)
    for k in range(_NG):
        sl = pl.ds(k * 16, 16)
        cellm = cells[k, :]
        locid = k * 16 + lane
        valid = locid < _T
        got = plsc.load_gather(table, [cellm])
        win = valid & (got == locid + 1)
        meta[0, sl] = jnp.where(win, 1.0, 0.0)
    gcp.wait()
    pltpu.sync_copy(rows, rows_out.at[b])
    pltpu.sync_copy(meta, meta_out.at[b])


def _tc_body(rows_ref, meta_ref, out_ref):
    rows = rows_ref[...]
    meta = meta_ref[...]
    win = meta[0, :] > 0.5
    txv = meta[1, :]
    tyv = meta[2, :]
    rw = meta[3, :]
    rh = meta[4, :]
    clsv = meta[5, :].astype(jnp.int32)
    coord = ((rows[:, 0] - txv) ** 2 + (rows[:, 1] - tyv) ** 2
             + (rows[:, 2] - jnp.log(rw)) ** 2
             + (rows[:, 3] - jnp.log(rh)) ** 2)
    p = jnp.clip(rows[:, 5:_D], 1e-07, 1.0 - 1e-07)
    ch = lax.broadcasted_iota(jnp.int32, p.shape, 1)
    onehot = ch == clsv[:, None]
    lc_row = jnp.sum(jnp.where(onehot, -jnp.log(p), -jnp.log(1.0 - p)),
                     axis=1)
    cnt = jnp.maximum(jnp.sum(jnp.where(win, 1.0, 0.0)), 1.0)
    tot = jnp.sum(jnp.where(win, coord + lc_row / float(_C), 0.0))
    out_ref[...] = jnp.reshape(tot / cnt, (1, 1))


_tc_loss = pl.pallas_call(
    _tc_body,
    out_shape=jax.ShapeDtypeStruct((1, 1), jnp.float32),
)


def kernel(output, anchors, targets):
    outflat = jnp.pad(output.reshape(_NROW, _D), ((0, 0), (0, 128 - _D)))
    tt = jnp.transpose(targets, (0, 2, 1))
    tprep = jnp.concatenate(
        [tt, jnp.zeros((_B, 5, _TPAD - _T), jnp.float32)], axis=2)
    ancrep = jnp.broadcast_to(anchors.reshape(6, 1), (6, 16))
    rows_out, meta_out = _sc_stage(outflat, tprep, ancrep)
    rows2d = rows_out.reshape(_B * _TPAD, 128)
    meta2 = jnp.transpose(meta_out, (1, 0, 2)).reshape(8, _B * _TPAD)
    loss = _tc_loss(rows2d, meta2)
    return loss[0, 0]


# speedup vs baseline: 10.1877x; 2.6177x over previous
import functools

import jax
import jax.numpy as jnp
from jax import lax
from jax.experimental import pallas as pl
from jax.experimental.pallas import tpu as pltpu
from jax.experimental.pallas import tpu_sc as plsc

_B, _A, _G, _C, _T = 32, 3, 52, 80, 100
_D = 5 + _C
_NCELL = _A * _G * _G
_TBL = 8192
_TPAD = 112
_NG = _TPAD // 16
_NC, _NS = 2, 16
_NROW = _B * _A * _G * _G

_mesh = plsc.VectorSubcoreMesh(
    core_axis_name="c", subcore_axis_name="s",
    num_cores=_NC, num_subcores=_NS)


@functools.partial(
    pl.kernel,
    out_type=(jax.ShapeDtypeStruct((_B, _TPAD, _D), jnp.float32),
              jax.ShapeDtypeStruct((_B, 8, _TPAD), jnp.float32)),
    mesh=_mesh,
    compiler_params=pltpu.CompilerParams(needs_layout_passes=False),
    scratch_types=(
        pltpu.VMEM((5, _TPAD), jnp.float32),
        pltpu.VMEM((6, 16), jnp.float32),
        pltpu.VMEM((_TBL,), jnp.int32),
        pltpu.VMEM((_NG, 16), jnp.int32),
        pltpu.VMEM((_TPAD, _D), jnp.float32),
        pltpu.VMEM((8, _TPAD), jnp.float32),
        pltpu.SemaphoreType.DMA,
    ),
)
def _sc_stage(outview, tprep, ancrep, rows_out, meta_out,
              tloc, anc, table, cells, rows, meta, sem):
    b = lax.axis_index("s") * _NC + lax.axis_index("c")
    pltpu.sync_copy(tprep.at[b], tloc)
    pltpu.sync_copy(ancrep, anc)

    def _zero(i, carry):
        for u in range(8):
            table[pl.ds(i * 128 + u * 16, 16)] = jnp.zeros((16,), jnp.int32)
        return carry
    lax.fori_loop(0, _TBL // 128, _zero, 0)

    lane = lax.iota(jnp.int32, 16)
    aw = [anc[2 * a, :] for a in range(_A)]
    ah = [anc[2 * a + 1, :] for a in range(_A)]

    cps = []
    for k in range(_NG):
        sl = pl.ds(k * 16, 16)
        t0 = tloc[0, sl]
        t1 = tloc[1, sl]
        t2 = tloc[2, sl]
        t3 = tloc[3, sl]
        t4 = tloc[4, sl]
        gxf = t0 * float(_G)
        gyf = t1 * float(_G)
        gx = gxf.astype(jnp.int32)
        gy = gyf.astype(jnp.int32)
        fx = gxf - gx.astype(jnp.float32)
        fy = gyf - gy.astype(jnp.float32)
        gw = jnp.abs(t2 - t0) * float(_G)
        gh = jnp.abs(t3 - t1) * float(_G)
        gprod = gw * gh
        best = jnp.zeros((16,), jnp.int32)
        bw = aw[0]
        bh = ah[0]
        inter = jnp.minimum(aw[0], gw) * jnp.minimum(ah[0], gh)
        biou = inter / (1e-08 + aw[0] * ah[0] + gprod - inter)
        for a in range(1, _A):
            inter = jnp.minimum(aw[a], gw) * jnp.minimum(ah[a], gh)
            iou = inter / (1e-08 + aw[a] * ah[a] + gprod - inter)
            upd = iou > biou
            best = jnp.where(upd, a, best)
            bw = jnp.where(upd, aw[a], bw)
            bh = jnp.where(upd, ah[a], bh)
            biou = jnp.maximum(biou, iou)

        locid = k * 16 + lane
        valid = locid < _T
        cell = (best * _G + gx) * _G + gy
        cellm = jnp.where(valid, cell, _NCELL + lane)
        cells[k, :] = cellm
        meta[1, sl] = fx
        meta[2, sl] = fy
        meta[3, sl] = jnp.where(valid, gw / bw, 1.0)
        meta[4, sl] = jnp.where(valid, gh / bh, 1.0)
        meta[5, sl] = t4.astype(jnp.int32).astype(jnp.float32)

        ival = jnp.where(valid, locid + 1, 0)
        skey, sperm = plsc.sort_key_val(cellm * 16 + lane, lane)
        scell = lax.shift_right_logical(skey, 4)
        _, nxt = plsc.sort_key_val((lane + 15) & 15, scell)
        rep_sorted = jnp.where((scell != nxt) | (lane == 15), 1, 0)
        _, rep = plsc.sort_key_val(sperm, rep_sorted)
        old = plsc.load_gather(table, [cellm])
        plsc.store_scatter(table, [cellm], jnp.maximum(old, ival),
                           mask=valid & (rep > 0))

        item = (b * _A + best) * _G + gx
        for l in range(16):
            if k * 16 + l < _T:
                cps.append(pltpu.async_copy(
                    outview.at[item[l], gy[l]], rows.at[k * 16 + l], sem))

    for k in range(_NG):
        sl = pl.ds(k * 16, 16)
        cellm = cells[k, :]
        locid = k * 16 + lane
        valid = locid < _T
        got = plsc.load_gather(table, [cellm])
        win = valid & (got == locid + 1)
        meta[0, sl] = jnp.where(win, 1.0, 0.0)
    for cp in cps:
        cp.wait()
    pltpu.sync_copy(rows, rows_out.at[b])
    pltpu.sync_copy(meta, meta_out.at[b])


def _tc_body(rows_ref, meta_ref, out_ref):
    rows = rows_ref[...]
    meta = meta_ref[...]
    win = meta[0, :] > 0.5
    txv = meta[1, :]
    tyv = meta[2, :]
    rw = meta[3, :]
    rh = meta[4, :]
    clsv = meta[5, :].astype(jnp.int32)
    coord = ((rows[:, 0] - txv) ** 2 + (rows[:, 1] - tyv) ** 2
             + (rows[:, 2] - jnp.log(rw)) ** 2
             + (rows[:, 3] - jnp.log(rh)) ** 2)
    p = jnp.clip(rows[:, 5:_D], 1e-07, 1.0 - 1e-07)
    ch = lax.broadcasted_iota(jnp.int32, p.shape, 1)
    onehot = ch == clsv[:, None]
    lc_row = jnp.sum(jnp.where(onehot, -jnp.log(p), -jnp.log(1.0 - p)),
                     axis=1)
    cnt = jnp.maximum(jnp.sum(jnp.where(win, 1.0, 0.0)), 1.0)
    tot = jnp.sum(jnp.where(win, coord + lc_row / float(_C), 0.0))
    out_ref[...] = jnp.reshape(tot / cnt, (1, 1))


_tc_loss = pl.pallas_call(
    _tc_body,
    out_shape=jax.ShapeDtypeStruct((1, 1), jnp.float32),
)


def kernel(output, anchors, targets):
    outview = output.reshape(_B * _A * _G, _G, _D)
    tt = jnp.transpose(targets, (0, 2, 1))
    tprep = jnp.concatenate(
        [tt, jnp.zeros((_B, 5, _TPAD - _T), jnp.float32)], axis=2)
    ancrep = jnp.broadcast_to(anchors.reshape(6, 1), (6, 16))
    rows_out, meta_out = _sc_stage(outview, tprep, ancrep)
    rows2d = rows_out.reshape(_B * _TPAD, _D)
    meta2 = jnp.transpose(meta_out, (1, 0, 2)).reshape(8, _B * _TPAD)
    loss = _tc_loss(rows2d, meta2)
    return loss[0, 0]


# speedup vs baseline: 11.9850x; 1.1764x over previous
import functools

import jax
import jax.numpy as jnp
from jax import lax
from jax.experimental import pallas as pl
from jax.experimental.pallas import tpu as pltpu
from jax.experimental.pallas import tpu_sc as plsc

_B, _A, _G, _C, _T = 32, 3, 52, 80, 100
_D = 5 + _C
_NCELL = _A * _G * _G
_TBL = 8192
_TPAD = 112
_NG = _TPAD // 16
_NC, _NS = 2, 16
_NROW = _B * _A * _G * _G

_mesh = plsc.VectorSubcoreMesh(
    core_axis_name="c", subcore_axis_name="s",
    num_cores=_NC, num_subcores=_NS)


@functools.partial(
    pl.kernel,
    out_type=(jax.ShapeDtypeStruct((_B, _TPAD, _D), jnp.float32),
              jax.ShapeDtypeStruct((_B, 8, _TPAD), jnp.float32)),
    mesh=_mesh,
    compiler_params=pltpu.CompilerParams(needs_layout_passes=False),
    scratch_types=(
        pltpu.VMEM((5, _TPAD), jnp.float32),
        pltpu.VMEM((6, 16), jnp.float32),
        pltpu.VMEM((_TBL,), jnp.int32),
        pltpu.VMEM((_NG, 16), jnp.int32),
        pltpu.VMEM((_TPAD, _D), jnp.float32),
        pltpu.VMEM((8, _TPAD), jnp.float32),
        pltpu.SemaphoreType.DMA,
    ),
)
def _sc_stage(outview, tprep, ancrep, rows_out, meta_out,
              tloc, anc, table, cells, rows, meta, sem):
    b = lax.axis_index("s") * _NC + lax.axis_index("c")
    pltpu.sync_copy(tprep.at[b], tloc)
    pltpu.sync_copy(ancrep, anc)

    def _zero(i, carry):
        for u in range(8):
            table[pl.ds(i * 128 + u * 16, 16)] = jnp.zeros((16,), jnp.int32)
        return carry
    lax.fori_loop(0, _TBL // 128, _zero, 0)

    lane = lax.iota(jnp.int32, 16)
    aw = [anc[2 * a, :] for a in range(_A)]
    ah = [anc[2 * a + 1, :] for a in range(_A)]

    cps = []
    for k in range(_NG):
        sl = pl.ds(k * 16, 16)
        t0 = tloc[0, sl]
        t1 = tloc[1, sl]
        t2 = tloc[2, sl]
        t3 = tloc[3, sl]
        t4 = tloc[4, sl]
        gxf = t0 * float(_G)
        gyf = t1 * float(_G)
        gx = gxf.astype(jnp.int32)
        gy = gyf.astype(jnp.int32)
        fx = gxf - gx.astype(jnp.float32)
        fy = gyf - gy.astype(jnp.float32)
        gw = jnp.abs(t2 - t0) * float(_G)
        gh = jnp.abs(t3 - t1) * float(_G)
        gprod = gw * gh
        best = jnp.zeros((16,), jnp.int32)
        bw = aw[0]
        bh = ah[0]
        inter = jnp.minimum(aw[0], gw) * jnp.minimum(ah[0], gh)
        biou = inter / (1e-08 + aw[0] * ah[0] + gprod - inter)
        for a in range(1, _A):
            inter = jnp.minimum(aw[a], gw) * jnp.minimum(ah[a], gh)
            iou = inter / (1e-08 + aw[a] * ah[a] + gprod - inter)
            upd = iou > biou
            best = jnp.where(upd, a, best)
            bw = jnp.where(upd, aw[a], bw)
            bh = jnp.where(upd, ah[a], bh)
            biou = jnp.maximum(biou, iou)

        locid = k * 16 + lane
        valid = locid < _T
        cell = (best * _G + gx) * _G + gy
        cellm = jnp.where(valid, cell, _NCELL + lane)
        cells[k, :] = cellm
        meta[1, sl] = fx
        meta[2, sl] = fy
        meta[3, sl] = jnp.where(valid, gw / bw, 1.0)
        meta[4, sl] = jnp.where(valid, gh / bh, 1.0)
        meta[5, sl] = t4.astype(jnp.int32).astype(jnp.float32)

        ival = jnp.where(valid, locid + 1, 0)
        skey, sperm = plsc.sort_key_val(cellm * 16 + lane, lane)
        scell = lax.shift_right_logical(skey, 4)
        _, nxt = plsc.sort_key_val((lane + 15) & 15, scell)
        rep_sorted = jnp.where((scell != nxt) | (lane == 15), 1, 0)
        _, rep = plsc.sort_key_val(sperm, rep_sorted)
        old = plsc.load_gather(table, [cellm])
        plsc.store_scatter(table, [cellm], jnp.maximum(old, ival),
                           mask=valid & (rep > 0))

        for l in range(16):
            if k * 16 + l < _T:
                cps.append(pltpu.async_copy(
                    outview.at[b, best[l], gx[l], gy[l]],
                    rows.at[k * 16 + l], sem))

    for k in range(_NG):
        sl = pl.ds(k * 16, 16)
        cellm = cells[k, :]
        locid = k * 16 + lane
        valid = locid < _T
        got = plsc.load_gather(table, [cellm])
        win = valid & (got == locid + 1)
        meta[0, sl] = jnp.where(win, 1.0, 0.0)
    for cp in cps:
        cp.wait()
    pltpu.sync_copy(rows, rows_out.at[b])
    pltpu.sync_copy(meta, meta_out.at[b])


def _tc_body(rows_ref, meta_ref, out_ref):
    rows = rows_ref[...]
    meta = meta_ref[...]
    win = meta[0, :] > 0.5
    txv = meta[1, :]
    tyv = meta[2, :]
    rw = meta[3, :]
    rh = meta[4, :]
    clsv = meta[5, :].astype(jnp.int32)
    coord = ((rows[:, 0] - txv) ** 2 + (rows[:, 1] - tyv) ** 2
             + (rows[:, 2] - jnp.log(rw)) ** 2
             + (rows[:, 3] - jnp.log(rh)) ** 2)
    p = jnp.clip(rows[:, 5:_D], 1e-07, 1.0 - 1e-07)
    ch = lax.broadcasted_iota(jnp.int32, p.shape, 1)
    onehot = ch == clsv[:, None]
    lc_row = jnp.sum(jnp.where(onehot, -jnp.log(p), -jnp.log(1.0 - p)),
                     axis=1)
    cnt = jnp.maximum(jnp.sum(jnp.where(win, 1.0, 0.0)), 1.0)
    tot = jnp.sum(jnp.where(win, coord + lc_row / float(_C), 0.0))
    out_ref[...] = jnp.reshape(tot / cnt, (1, 1))


_tc_loss = pl.pallas_call(
    _tc_body,
    out_shape=jax.ShapeDtypeStruct((1, 1), jnp.float32),
)


def kernel(output, anchors, targets):
    tt = jnp.transpose(targets, (0, 2, 1))
    tprep = jnp.concatenate(
        [tt, jnp.zeros((_B, 5, _TPAD - _T), jnp.float32)], axis=2)
    ancrep = jnp.broadcast_to(anchors.reshape(6, 1), (6, 16))
    rows_out, meta_out = _sc_stage(output, tprep, ancrep)
    rows2d = rows_out.reshape(_B * _TPAD, _D)
    meta2 = jnp.transpose(meta_out, (1, 0, 2)).reshape(8, _B * _TPAD)
    loss = _tc_loss(rows2d, meta2)
    return loss[0, 0]


# speedup vs baseline: 39.1409x; 3.2658x over previous
import functools

import jax
import jax.numpy as jnp
from jax import lax
from jax.experimental import pallas as pl
from jax.experimental.pallas import tpu as pltpu
from jax.experimental.pallas import tpu_sc as plsc

_B, _A, _G, _C, _T = 32, 3, 52, 80, 100
_D = 5 + _C
_NCELL = _A * _G * _G
_TBL = 8192
_TPAD = 112
_NG = _TPAD // 16
_NC, _NS = 2, 16
_NROW = _B * _A * _G * _G

_mesh = plsc.VectorSubcoreMesh(
    core_axis_name="c", subcore_axis_name="s",
    num_cores=_NC, num_subcores=_NS)


@functools.partial(
    pl.kernel,
    out_type=(jax.ShapeDtypeStruct((_B, _TPAD, _D), jnp.float32),
              jax.ShapeDtypeStruct((_B, 8, _TPAD), jnp.float32)),
    mesh=_mesh,
    compiler_params=pltpu.CompilerParams(needs_layout_passes=False),
    scratch_types=(
        pltpu.VMEM((5, _TPAD), jnp.float32),
        pltpu.VMEM((6, 16), jnp.float32),
        pltpu.VMEM((_TBL,), jnp.int32),
        pltpu.VMEM((_NG, 16), jnp.int32),
        pltpu.VMEM((_TPAD, _D), jnp.float32),
        pltpu.VMEM((8, _TPAD), jnp.float32),
        pltpu.SemaphoreType.DMA,
    ),
)
def _sc_stage(outview, tprep, ancrep, rows_out, meta_out,
              tloc, anc, table, cells, rows, meta, sem):
    b = lax.axis_index("s") * _NC + lax.axis_index("c")
    pltpu.sync_copy(tprep.at[b], tloc)
    pltpu.sync_copy(ancrep, anc)

    def _zero(i, carry):
        for u in range(8):
            table[pl.ds(i * 128 + u * 16, 16)] = jnp.zeros((16,), jnp.int32)
        return carry
    lax.fori_loop(0, _TBL // 128, _zero, 0)

    lane = lax.iota(jnp.int32, 16)
    aw = [anc[2 * a, :] for a in range(_A)]
    ah = [anc[2 * a + 1, :] for a in range(_A)]

    cps = []
    for k in range(_NG):
        sl = pl.ds(k * 16, 16)
        t0 = tloc[0, sl]
        t1 = tloc[1, sl]
        t2 = tloc[2, sl]
        t3 = tloc[3, sl]
        t4 = tloc[4, sl]
        gxf = t0 * float(_G)
        gyf = t1 * float(_G)
        gx = gxf.astype(jnp.int32)
        gy = gyf.astype(jnp.int32)
        fx = gxf - gx.astype(jnp.float32)
        fy = gyf - gy.astype(jnp.float32)
        gw = jnp.abs(t2 - t0) * float(_G)
        gh = jnp.abs(t3 - t1) * float(_G)
        gprod = gw * gh
        best = jnp.zeros((16,), jnp.int32)
        bw = aw[0]
        bh = ah[0]
        inter = jnp.minimum(aw[0], gw) * jnp.minimum(ah[0], gh)
        biou = inter / (1e-08 + aw[0] * ah[0] + gprod - inter)
        for a in range(1, _A):
            inter = jnp.minimum(aw[a], gw) * jnp.minimum(ah[a], gh)
            iou = inter / (1e-08 + aw[a] * ah[a] + gprod - inter)
            upd = iou > biou
            best = jnp.where(upd, a, best)
            bw = jnp.where(upd, aw[a], bw)
            bh = jnp.where(upd, ah[a], bh)
            biou = jnp.maximum(biou, iou)

        locid = k * 16 + lane
        valid = locid < _T
        cell = (best * _G + gx) * _G + gy
        cellm = jnp.where(valid, cell, _NCELL + lane)
        cells[k, :] = cellm
        meta[1, sl] = fx
        meta[2, sl] = fy
        meta[3, sl] = jnp.where(valid, gw / bw, 1.0)
        meta[4, sl] = jnp.where(valid, gh / bh, 1.0)
        meta[5, sl] = t4.astype(jnp.int32).astype(jnp.float32)

        ival = jnp.where(valid, locid + 1, 0)
        skey, sperm = plsc.sort_key_val(cellm * 16 + lane, lane)
        scell = lax.shift_right_logical(skey, 4)
        _, nxt = plsc.sort_key_val((lane + 15) & 15, scell)
        rep_sorted = jnp.where((scell != nxt) | (lane == 15), 1, 0)
        _, rep = plsc.sort_key_val(sperm, rep_sorted)
        old = plsc.load_gather(table, [cellm])
        plsc.store_scatter(table, [cellm], jnp.maximum(old, ival),
                           mask=valid & (rep > 0))

        for l in range(16):
            if k * 16 + l < _T:
                cps.append(pltpu.async_copy(
                    outview.at[best[l], gx[l], gy[l], b],
                    rows.at[k * 16 + l], sem))

    for k in range(_NG):
        sl = pl.ds(k * 16, 16)
        cellm = cells[k, :]
        locid = k * 16 + lane
        valid = locid < _T
        got = plsc.load_gather(table, [cellm])
        win = valid & (got == locid + 1)
        meta[0, sl] = jnp.where(win, 1.0, 0.0)
    for cp in cps:
        cp.wait()
    pltpu.sync_copy(rows, rows_out.at[b])
    pltpu.sync_copy(meta, meta_out.at[b])


def _tc_body(rows_ref, meta_ref, out_ref):
    rows = rows_ref[...]
    meta = meta_ref[...]
    win = meta[0, :] > 0.5
    txv = meta[1, :]
    tyv = meta[2, :]
    rw = meta[3, :]
    rh = meta[4, :]
    clsv = meta[5, :].astype(jnp.int32)
    coord = ((rows[:, 0] - txv) ** 2 + (rows[:, 1] - tyv) ** 2
             + (rows[:, 2] - jnp.log(rw)) ** 2
             + (rows[:, 3] - jnp.log(rh)) ** 2)
    p = jnp.clip(rows[:, 5:_D], 1e-07, 1.0 - 1e-07)
    ch = lax.broadcasted_iota(jnp.int32, p.shape, 1)
    onehot = ch == clsv[:, None]
    lc_row = jnp.sum(jnp.where(onehot, -jnp.log(p), -jnp.log(1.0 - p)),
                     axis=1)
    cnt = jnp.maximum(jnp.sum(jnp.where(win, 1.0, 0.0)), 1.0)
    tot = jnp.sum(jnp.where(win, coord + lc_row / float(_C), 0.0))
    out_ref[...] = jnp.reshape(tot / cnt, (1, 1))


_tc_loss = pl.pallas_call(
    _tc_body,
    out_shape=jax.ShapeDtypeStruct((1, 1), jnp.float32),
)


def kernel(output, anchors, targets):
    outt = jnp.transpose(output, (1, 2, 3, 0, 4))
    tt = jnp.transpose(targets, (0, 2, 1))
    tprep = jnp.concatenate(
        [tt, jnp.zeros((_B, 5, _TPAD - _T), jnp.float32)], axis=2)
    ancrep = jnp.broadcast_to(anchors.reshape(6, 1), (6, 16))
    rows_out, meta_out = _sc_stage(outt, tprep, ancrep)
    rows2d = rows_out.reshape(_B * _TPAD, _D)
    meta2 = jnp.transpose(meta_out, (1, 0, 2)).reshape(8, _B * _TPAD)
    loss = _tc_loss(rows2d, meta2)
    return loss[0, 0]


# speedup vs baseline: 44.1654x; 1.1284x over previous
import functools

import jax
import jax.numpy as jnp
from jax import lax
from jax.experimental import pallas as pl
from jax.experimental.pallas import tpu as pltpu
from jax.experimental.pallas import tpu_sc as plsc

_B, _A, _G, _C, _T = 32, 3, 52, 80, 100
_D = 5 + _C
_NCELL = _A * _G * _G
_TBL = 8192
_TPAD = 112
_NG = _TPAD // 16
_NC, _NS = 2, 16
_LN2 = 0.6931471805599453
_LOG2C = (0.043428363331612846, -0.40486230941594464, 1.5938845482689363,
          -3.4924660425574374, 5.046852935530177, -2.7868055642996286)

_mesh = plsc.VectorSubcoreMesh(
    core_axis_name="c", subcore_axis_name="s",
    num_cores=_NC, num_subcores=_NS)


def _vlog(x):
    bits = plsc.bitcast(x, jnp.int32)
    e = lax.shift_right_logical(bits, 23) - 127
    m = plsc.bitcast((bits & 0x007FFFFF) | 0x3F800000, jnp.float32)
    p = jnp.full_like(m, _LOG2C[0])
    for c in _LOG2C[1:]:
        p = p * m + c
    return (e.astype(jnp.float32) + p) * _LN2


@functools.partial(
    pl.kernel,
    out_type=jax.ShapeDtypeStruct((_B, 16), jnp.float32),
    mesh=_mesh,
    compiler_params=pltpu.CompilerParams(needs_layout_passes=False),
    scratch_types=(
        pltpu.VMEM((5, _TPAD), jnp.float32),
        pltpu.VMEM((6, 16), jnp.float32),
        pltpu.VMEM((_TBL,), jnp.int32),
        pltpu.VMEM((_NG, 16), jnp.int32),
        pltpu.VMEM((_TPAD, _D), jnp.float32),
        pltpu.VMEM((8, 128), jnp.float32),
        pltpu.VMEM((2, 16), jnp.float32),
        pltpu.VMEM((16,), jnp.float32),
        pltpu.SemaphoreType.DMA,
    ),
)
def _sc_stage(outview, tprep, ancrep, parts_out,
              tloc, anc, table, cells, rows, meta, acc, resv, sem):
    b = lax.axis_index("s") * _NC + lax.axis_index("c")
    pltpu.sync_copy(tprep.at[b], tloc)
    pltpu.sync_copy(ancrep, anc)

    def _zero(i, carry):
        for u in range(8):
            table[pl.ds(i * 128 + u * 16, 16)] = jnp.zeros((16,), jnp.int32)
        return carry
    lax.fori_loop(0, _TBL // 128, _zero, 0)

    lane = lax.iota(jnp.int32, 16)
    aw = [anc[2 * a, :] for a in range(_A)]
    ah = [anc[2 * a + 1, :] for a in range(_A)]

    cps = []
    for k in range(_NG):
        sl = pl.ds(k * 16, 16)
        t0 = tloc[0, sl]
        t1 = tloc[1, sl]
        t2 = tloc[2, sl]
        t3 = tloc[3, sl]
        t4 = tloc[4, sl]
        gxf = t0 * float(_G)
        gyf = t1 * float(_G)
        gx = gxf.astype(jnp.int32)
        gy = gyf.astype(jnp.int32)
        fx = gxf - gx.astype(jnp.float32)
        fy = gyf - gy.astype(jnp.float32)
        gw = jnp.abs(t2 - t0) * float(_G)
        gh = jnp.abs(t3 - t1) * float(_G)
        gprod = gw * gh
        best = jnp.zeros((16,), jnp.int32)
        bw = aw[0]
        bh = ah[0]
        inter = jnp.minimum(aw[0], gw) * jnp.minimum(ah[0], gh)
        biou = inter / (1e-08 + aw[0] * ah[0] + gprod - inter)
        for a in range(1, _A):
            inter = jnp.minimum(aw[a], gw) * jnp.minimum(ah[a], gh)
            iou = inter / (1e-08 + aw[a] * ah[a] + gprod - inter)
            upd = iou > biou
            best = jnp.where(upd, a, best)
            bw = jnp.where(upd, aw[a], bw)
            bh = jnp.where(upd, ah[a], bh)
            biou = jnp.maximum(biou, iou)

        locid = k * 16 + lane
        valid = locid < _T
        cell = (best * _G + gx) * _G + gy
        cellm = jnp.where(valid, cell, _NCELL + lane)
        cells[k, :] = cellm
        meta[1, sl] = fx
        meta[2, sl] = fy
        meta[3, sl] = _vlog(jnp.where(valid, gw / bw, 1.0))
        meta[4, sl] = _vlog(jnp.where(valid, gh / bh, 1.0))
        meta[5, sl] = t4.astype(jnp.int32).astype(jnp.float32)

        ival = jnp.where(valid, locid + 1, 0)
        skey, sperm = plsc.sort_key_val(cellm * 16 + lane, lane)
        scell = lax.shift_right_logical(skey, 4)
        _, nxt = plsc.sort_key_val((lane + 15) & 15, scell)
        rep_sorted = jnp.where((scell != nxt) | (lane == 15), 1, 0)
        _, rep = plsc.sort_key_val(sperm, rep_sorted)
        old = plsc.load_gather(table, [cellm])
        plsc.store_scatter(table, [cellm], jnp.maximum(old, ival),
                           mask=valid & (rep > 0))

        for l in range(16):
            if k * 16 + l < _T:
                cps.append(pltpu.async_copy(
                    outview.at[best[l], gx[l], gy[l], b],
                    rows.at[k * 16 + l], sem))

    wsumv = jnp.zeros((16,), jnp.float32)
    for k in range(_NG):
        sl = pl.ds(k * 16, 16)
        cellm = cells[k, :]
        locid = k * 16 + lane
        valid = locid < _T
        got = plsc.load_gather(table, [cellm])
        win = valid & (got == locid + 1)
        winf = jnp.where(win, 1.0, 0.0)
        meta[0, sl] = winf
        wsumv = wsumv + winf
    for cp in cps:
        cp.wait()

    acc[0, :] = jnp.zeros((16,), jnp.float32)
    acc[1, :] = jnp.zeros((16,), jnp.float32)

    def _row(i, carry):
        win0 = meta[0, pl.ds(i, 16)][0]
        tx0 = meta[1, pl.ds(i, 16)][0]
        ty0 = meta[2, pl.ds(i, 16)][0]
        lw0 = meta[3, pl.ds(i, 16)][0]
        lh0 = meta[4, pl.ds(i, 16)][0]
        cls0 = meta[5, pl.ds(i, 16)][0].astype(jnp.int32)
        v = rows[i, pl.ds(0, 16)]
        tgt = jnp.where(lane == 0, tx0,
                        jnp.where(lane == 1, ty0,
                                  jnp.where(lane == 2, lw0, lh0)))
        d = v - tgt
        acc[0, :] = acc[0, :] + jnp.where(lane < 4, d * d, 0.0) * win0
        s = jnp.zeros((16,), jnp.float32)
        for c in range(5):
            ch = rows[i, pl.ds(5 + 16 * c, 16)]
            pcl = jnp.clip(ch, 1e-07, 1.0 - 1e-07)
            qcl = jnp.clip(1.0 - ch, 1e-07, 1.0 - 1e-07)
            val = jnp.where(lane + 16 * c == cls0, pcl, qcl)
            s = s - _vlog(val)
        acc[1, :] = acc[1, :] + s * win0
        return carry

    lax.fori_loop(0, _T, _row, 0)

    csum = jnp.sum(acc[0, :])
    bsum = jnp.sum(acc[1, :])
    wsum = jnp.sum(wsumv)
    resv[...] = jnp.where(lane == 0, csum,
                          jnp.where(lane == 1, bsum,
                                    jnp.where(lane == 2, wsum, 0.0)))
    pltpu.sync_copy(resv, parts_out.at[b])


def _tc_body(parts_ref, out_ref):
    p = parts_ref[...]
    csum = jnp.sum(p[:, 0])
    bsum = jnp.sum(p[:, 1])
    cnt = jnp.maximum(jnp.sum(p[:, 2]), 1.0)
    out_ref[...] = jnp.reshape((csum + bsum / float(_C)) / cnt, (1, 1))


_tc_reduce = pl.pallas_call(
    _tc_body,
    out_shape=jax.ShapeDtypeStruct((1, 1), jnp.float32),
)


def kernel(output, anchors, targets):
    outt = jnp.transpose(output, (1, 2, 3, 0, 4))
    tt = jnp.transpose(targets, (0, 2, 1))
    tprep = jnp.concatenate(
        [tt, jnp.zeros((_B, 5, _TPAD - _T), jnp.float32)], axis=2)
    ancrep = jnp.broadcast_to(anchors.reshape(6, 1), (6, 16))
    parts = _sc_stage(outt, tprep, ancrep)
    loss = _tc_reduce(parts)
    return loss[0, 0]


# speedup vs baseline: 44.7540x; 1.0133x over previous
import functools

import jax
import jax.numpy as jnp
from jax import lax
from jax.experimental import pallas as pl
from jax.experimental.pallas import tpu as pltpu
from jax.experimental.pallas import tpu_sc as plsc

_B, _A, _G, _C, _T = 32, 3, 52, 80, 100
_D = 5 + _C
_NCELL = _A * _G * _G
_TBL = 8192
_TPAD = 112
_NG = _TPAD // 16
_NC, _NS = 2, 16
_LN2 = 0.6931471805599453
_LOG2C = (0.043428363331612846, -0.40486230941594464, 1.5938845482689363,
          -3.4924660425574374, 5.046852935530177, -2.7868055642996286)

_mesh = plsc.VectorSubcoreMesh(
    core_axis_name="c", subcore_axis_name="s",
    num_cores=_NC, num_subcores=_NS)


def _vlog(x):
    bits = plsc.bitcast(x, jnp.int32)
    e = lax.shift_right_logical(bits, 23) - 127
    m = plsc.bitcast((bits & 0x007FFFFF) | 0x3F800000, jnp.float32)
    p = jnp.full_like(m, _LOG2C[0])
    for c in _LOG2C[1:]:
        p = p * m + c
    return (e.astype(jnp.float32) + p) * _LN2


@functools.partial(
    pl.kernel,
    out_type=jax.ShapeDtypeStruct((_B, 16), jnp.float32),
    mesh=_mesh,
    compiler_params=pltpu.CompilerParams(needs_layout_passes=False),
    scratch_types=(
        pltpu.VMEM((5, _TPAD), jnp.float32),
        pltpu.VMEM((6, 16), jnp.float32),
        pltpu.VMEM((_TBL,), jnp.int32),
        pltpu.VMEM((_NG, 16), jnp.int32),
        pltpu.VMEM((_TPAD, _D), jnp.float32),
        pltpu.VMEM((8, 128), jnp.float32),
        pltpu.VMEM((2, 16), jnp.float32),
        pltpu.VMEM((16,), jnp.float32),
        pltpu.SemaphoreType.DMA,
    ),
)
def _sc_stage(outview, tprep, ancrep, parts_out,
              tloc, anc, table, cells, rows, meta, acc, resv, sem):
    b = lax.axis_index("s") * _NC + lax.axis_index("c")
    pltpu.sync_copy(tprep.at[b], tloc)
    pltpu.sync_copy(ancrep, anc)

    def _zero(i, carry):
        for u in range(8):
            table[pl.ds(i * 128 + u * 16, 16)] = jnp.zeros((16,), jnp.int32)
        return carry
    lax.fori_loop(0, _TBL // 128, _zero, 0)

    lane = lax.iota(jnp.int32, 16)
    aw = [anc[2 * a, :] for a in range(_A)]
    ah = [anc[2 * a + 1, :] for a in range(_A)]

    cps = []
    for k in range(_NG):
        sl = pl.ds(k * 16, 16)
        t0 = tloc[0, sl]
        t1 = tloc[1, sl]
        t2 = tloc[2, sl]
        t3 = tloc[3, sl]
        t4 = tloc[4, sl]
        gxf = t0 * float(_G)
        gyf = t1 * float(_G)
        gx = gxf.astype(jnp.int32)
        gy = gyf.astype(jnp.int32)
        fx = gxf - gx.astype(jnp.float32)
        fy = gyf - gy.astype(jnp.float32)
        gw = jnp.abs(t2 - t0) * float(_G)
        gh = jnp.abs(t3 - t1) * float(_G)
        gprod = gw * gh
        best = jnp.zeros((16,), jnp.int32)
        bw = aw[0]
        bh = ah[0]
        inter = jnp.minimum(aw[0], gw) * jnp.minimum(ah[0], gh)
        biou = inter / (1e-08 + aw[0] * ah[0] + gprod - inter)
        for a in range(1, _A):
            inter = jnp.minimum(aw[a], gw) * jnp.minimum(ah[a], gh)
            iou = inter / (1e-08 + aw[a] * ah[a] + gprod - inter)
            upd = iou > biou
            best = jnp.where(upd, a, best)
            bw = jnp.where(upd, aw[a], bw)
            bh = jnp.where(upd, ah[a], bh)
            biou = jnp.maximum(biou, iou)

        locid = k * 16 + lane
        valid = locid < _T
        cell = (best * _G + gx) * _G + gy
        cellm = jnp.where(valid, cell, _NCELL + lane)
        cells[k, :] = cellm
        meta[1, sl] = fx
        meta[2, sl] = fy
        meta[3, sl] = _vlog(jnp.where(valid, gw / bw, 1.0))
        meta[4, sl] = _vlog(jnp.where(valid, gh / bh, 1.0))
        meta[5, sl] = t4.astype(jnp.int32).astype(jnp.float32)

        ival = jnp.where(valid, locid + 1, 0)
        skey, sperm = plsc.sort_key_val(cellm * 16 + lane, lane)
        scell = lax.shift_right_logical(skey, 4)
        _, nxt = plsc.sort_key_val((lane + 15) & 15, scell)
        rep_sorted = jnp.where((scell != nxt) | (lane == 15), 1, 0)
        _, rep = plsc.sort_key_val(sperm, rep_sorted)
        old = plsc.load_gather(table, [cellm])
        plsc.store_scatter(table, [cellm], jnp.maximum(old, ival),
                           mask=valid & (rep > 0))

        for l in range(16):
            if k * 16 + l < _T:
                cps.append(pltpu.async_copy(
                    outview.at[best[l], gx[l], gy[l], b],
                    rows.at[k * 16 + l], sem))

    wsumv = jnp.zeros((16,), jnp.float32)
    for k in range(_NG):
        sl = pl.ds(k * 16, 16)
        cellm = cells[k, :]
        locid = k * 16 + lane
        valid = locid < _T
        got = plsc.load_gather(table, [cellm])
        win = valid & (got == locid + 1)
        winf = jnp.where(win, 1.0, 0.0)
        meta[0, sl] = winf
        wsumv = wsumv + winf
    for cp in cps:
        cp.wait()

    acc[0, :] = jnp.zeros((16,), jnp.float32)
    acc[1, :] = jnp.zeros((16,), jnp.float32)

    def _row(i, carry):
        win0 = meta[0, pl.ds(i, 16)][0]
        tx0 = meta[1, pl.ds(i, 16)][0]
        ty0 = meta[2, pl.ds(i, 16)][0]
        lw0 = meta[3, pl.ds(i, 16)][0]
        lh0 = meta[4, pl.ds(i, 16)][0]
        cls0 = meta[5, pl.ds(i, 16)][0].astype(jnp.int32)
        v = rows[i, pl.ds(0, 16)]
        tgt = jnp.where(lane == 0, tx0,
                        jnp.where(lane == 1, ty0,
                                  jnp.where(lane == 2, lw0, lh0)))
        d = v - tgt
        acc[0, :] = acc[0, :] + jnp.where(lane < 4, d * d, 0.0) * win0
        prod = jnp.full((16,), 1.0, jnp.float32)
        for c in range(5):
            ch = rows[i, pl.ds(5 + 16 * c, 16)]
            pcl = jnp.clip(ch, 1e-07, 1.0 - 1e-07)
            qcl = jnp.clip(1.0 - ch, 1e-07, 1.0 - 1e-07)
            prod = prod * jnp.where(lane + 16 * c == cls0, pcl, qcl)
        acc[1, :] = acc[1, :] - _vlog(prod) * win0
        return carry

    lax.fori_loop(0, _T, _row, 0)

    csum = jnp.sum(acc[0, :])
    bsum = jnp.sum(acc[1, :])
    wsum = jnp.sum(wsumv)
    resv[...] = jnp.where(lane == 0, csum,
                          jnp.where(lane == 1, bsum,
                                    jnp.where(lane == 2, wsum, 0.0)))
    pltpu.sync_copy(resv, parts_out.at[b])


def _tc_body(parts_ref, out_ref):
    p = parts_ref[...]
    csum = jnp.sum(p[:, 0])
    bsum = jnp.sum(p[:, 1])
    cnt = jnp.maximum(jnp.sum(p[:, 2]), 1.0)
    out_ref[...] = jnp.reshape((csum + bsum / float(_C)) / cnt, (1, 1))


_tc_reduce = pl.pallas_call(
    _tc_body,
    out_shape=jax.ShapeDtypeStruct((1, 1), jnp.float32),
)


def kernel(output, anchors, targets):
    outt = jnp.transpose(output, (1, 2, 3, 0, 4))
    tt = jnp.transpose(targets, (0, 2, 1))
    tprep = jnp.concatenate(
        [tt, jnp.zeros((_B, 5, _TPAD - _T), jnp.float32)], axis=2)
    ancrep = jnp.broadcast_to(anchors.reshape(6, 1), (6, 16))
    parts = _sc_stage(outt, tprep, ancrep)
    loss = _tc_reduce(parts)
    return loss[0, 0]
